# Initial kernel scaffold; baseline (speedup 1.0000x reference)
import jax
import jax.numpy as jnp
from jax.experimental import pallas as pl


def kernel(positions, plane_tx, plane_ty, plane_tz, plane_xy, plane_xz, plane_yz):
    raise NotImplementedError("write your pallas kernel here")


# speedup vs baseline: 38.3121x; 38.3121x over previous
import jax
import jax.numpy as jnp
from jax import lax
from jax.experimental import pallas as pl
from jax.experimental.pallas import tpu as pltpu
from jax.experimental.pallas import tpu_sc as plsc

F = 16
C = 16
NSITE = 129 * 129
D = F * C
B_TOTAL = 8192
NWORKERS = 32
PER_W = B_TOTAL // NWORKERS
CHUNK = 16
NCHUNKS = PER_W // CHUNK
CORNER_OFF = (0, 1, 129, 130)
PAIRS = ((0, 5), (1, 4), (2, 3))

_GATHER_DNUMS = lax.GatherDimensionNumbers(
    offset_dims=(), collapsed_slice_dims=(0,), start_index_map=(0,))


def _splat(vec, lane):
    idx = jnp.full((16,), lane, jnp.int32)
    return lax.gather(vec, idx[:, None], _GATHER_DNUMS, (1,),
                      mode=lax.GatherScatterMode.PROMISE_IN_BOUNDS)


def _body(pos_ref, t0, t1, t2, t3, t4, t5, out_ref,
          pos_v, bases, wmat, idx_v, dest, out_t, sem):
    tables = (t0, t1, t2, t3, t4, t5)
    wid = lax.axis_index("s") * 2 + lax.axis_index("c")
    base_pos = wid * PER_W

    for c4 in range(4):
        pltpu.sync_copy(pos_ref.at[c4, pl.ds(base_pos, PER_W)],
                        pos_v.at[pl.ds(c4 * PER_W, PER_W)])

    lane_f = lax.iota(jnp.int32, 16)
    bit_masks = [(lax.shift_right_logical(lane_f, 3 - k) & 1) == 1
                 for k in range(4)]

    def precompute(g, carry):
        pv = [pos_v[pl.ds(c4 * PER_W + g * 16, 16)] for c4 in range(4)]
        m = jnp.full((16,), 1.0, jnp.float32)
        igs, frs = [], []
        for c4 in range(4):
            p = pv[c4]
            ok = (p >= 0.0) & (p < 1.0)
            m = jnp.where(ok, m, 0.0)
            ps = p * 128.0
            ii = ps.astype(jnp.int32)
            ii = jnp.minimum(jnp.maximum(ii, 0), 127)
            frs.append(ps - ii.astype(jnp.float32))
            igs.append(ii)
        it, ix, iy, iz = igs
        t129 = it * 129
        x129 = ix * 129
        y129 = iy * 129
        site = (t129 + ix, t129 + iy, t129 + iz,
                x129 + iy, x129 + iz, y129 + iz)
        for p_i in range(6):
            bases[pl.ds(p_i * PER_W + g * 16, 16)] = site[p_i]
        for jj in range(16):
            w = _splat(m, jj)
            for c4 in range(4):
                s = _splat(frs[c4], jj)
                w = w * jnp.where(bit_masks[c4], s, 1.0 - s)
            wmat[pl.ds((g * 16 + jj) * 16, 16)] = w
        return carry

    lax.fori_loop(0, PER_W // 16, precompute, None, unroll=False)

    def chunk_body(ck, carry):
        for p_i in range(6):
            bvec = bases[pl.ds(p_i * PER_W + ck * 16, 16)]
            for k in range(4):
                idx_v[pl.ds(p_i * 64 + k * 16, 16)] = bvec + CORNER_OFF[k]
        handles = []
        for p_i in range(6):
            handles.append(pltpu.async_copy(
                tables[p_i].at[idx_v.at[pl.ds(p_i * 64, 64)]], dest.at[p_i], sem))
        for h in handles:
            h.wait()

        def pos_body(j, carry2):
            row = ck * 16 + j
            wv = wmat[pl.ds(row * 16, 16)]
            acc = jnp.zeros((16,), jnp.float32)
            for f in range(F):
                wf = _splat(wv, f)
                co = f * 16
                for (a_i, b_i) in PAIRS:
                    sa = ((dest[a_i, j, pl.ds(co, 16)]
                           + dest[a_i, j + 16, pl.ds(co, 16)])
                          + (dest[a_i, j + 32, pl.ds(co, 16)]
                             + dest[a_i, j + 48, pl.ds(co, 16)]))
                    sb = ((dest[b_i, j, pl.ds(co, 16)]
                           + dest[b_i, j + 16, pl.ds(co, 16)])
                          + (dest[b_i, j + 32, pl.ds(co, 16)]
                             + dest[b_i, j + 48, pl.ds(co, 16)]))
                    acc = acc + (sa * sb) * wf
            out_t[pl.ds(row * 16, 16)] = acc
            return carry2

        lax.fori_loop(0, CHUNK, pos_body, None, unroll=False)
        return carry

    lax.fori_loop(0, NCHUNKS, chunk_body, None, unroll=False)

    pltpu.sync_copy(out_t, out_ref.at[pl.ds(base_pos * C, PER_W * C)])


@jax.jit
def _run(pos_t, *tables):
    mesh = plsc.VectorSubcoreMesh(core_axis_name="c", subcore_axis_name="s")
    return pl.kernel(
        _body,
        out_type=jax.ShapeDtypeStruct((B_TOTAL * C,), jnp.float32),
        mesh=mesh,
        scratch_types=[
            pltpu.VMEM((4 * PER_W,), jnp.float32),
            pltpu.VMEM((6 * PER_W,), jnp.int32),
            pltpu.VMEM((PER_W * 16,), jnp.float32),
            pltpu.VMEM((6 * 64,), jnp.int32),
            pltpu.VMEM((6, 64, D), jnp.float32),
            pltpu.VMEM((PER_W * C,), jnp.float32),
            pltpu.SemaphoreType.DMA,
        ],
    )(pos_t, *tables)


def kernel(positions, plane_tx, plane_ty, plane_tz, plane_xy, plane_xz, plane_yz):
    pos_t = positions.T
    tables = [p.transpose(1, 2, 0, 3).reshape(NSITE, D)
              for p in (plane_tx, plane_ty, plane_tz, plane_xy, plane_xz, plane_yz)]
    return _run(pos_t, *tables).reshape(B_TOTAL, C)
